# Initial kernel scaffold; baseline (speedup 1.0000x reference)
import jax
import jax.numpy as jnp
from jax.experimental import pallas as pl


def kernel(z, edge_index, pos, batch, emb, e0_w1, e0_b1, e0_w2, e0_b2, e0_wi, e0_bi, n0_w1, n0_b1, n0_w2, n0_b2, e1_w1, e1_b1, e1_w2, e1_b2, e1_wi, e1_bi, n1_w1, n1_b1, n1_w2, n1_b2, o1_w1, o1_b1, o1_w2, o1_b2, o2_w1, o2_b1, o2_w2, o2_b2):
    raise NotImplementedError("write your pallas kernel here")


# speedup vs baseline: 3.1325x; 3.1325x over previous
import functools

import jax
import jax.numpy as jnp
from jax import lax
from jax.experimental import pallas as pl
from jax.experimental.pallas import tpu as pltpu
from jax.experimental.pallas import tpu_sc as plsc

F32 = jnp.float32
HID = 128
NN = 10000
NE = 320000
NG = 64
NC, NS = 2, 16
NW = NC * NS
EPW = NE // NW
C = 80
NCH = EPW // C
NCHP = 128
RPT = 624
RPT_LAST = NN - 15 * RPT
BE = 4000
BN = 2000
HI = jax.lax.Precision.HIGHEST

@functools.lru_cache(maxsize=None)
def _mesh():
    return plsc.VectorSubcoreMesh(
        core_axis_name="c", subcore_axis_name="s",
        num_cores=NC, num_subcores=NS)


def _sig(x):
    return 1.0 / (1.0 + jnp.exp(-x))


def _silu(x):
    return x * _sig(x)


@functools.lru_cache(maxsize=None)
def _make_sc_gather(with_d):
    out_type = [
        jax.ShapeDtypeStruct((NE, HID), F32),
        jax.ShapeDtypeStruct((NE, HID), F32),
    ]
    scratch = [
        pltpu.VMEM((NCHP, C), jnp.int32),
        pltpu.VMEM((NCHP, C), jnp.int32),
        pltpu.VMEM((C, HID), F32),
        pltpu.VMEM((C, HID), F32),
        pltpu.VMEM((C, HID), F32),
        pltpu.VMEM((C, HID), F32),
        pltpu.SemaphoreType.DMA,
        pltpu.SemaphoreType.DMA,
        pltpu.SemaphoreType.DMA,
        pltpu.SemaphoreType.DMA,
        pltpu.SemaphoreType.DMA,
        pltpu.SemaphoreType.DMA,
        pltpu.SemaphoreType.DMA,
        pltpu.SemaphoreType.DMA,
    ]
    if with_d:
        out_type.append(jax.ShapeDtypeStruct((NW * NCHP, C), F32))
        scratch += [
            pltpu.VMEM((NN,), F32),
            pltpu.VMEM((NN,), F32),
            pltpu.VMEM((NN,), F32),
            pltpu.VMEM((NCHP, C), F32),
        ]

    def body(*refs):
        if with_d:
            (a_hbm, b_hbm, recv4, send4, posx, posy, posz, ga_hbm, gb_hbm, d_hbm,
             idxr, idxs, a0, a1, b0, b1,
             sga0, sga1, sgb0, sgb1, swa0, swa1, swb0, swb1,
             px, py, pz, dall) = refs
        else:
            (a_hbm, b_hbm, recv4, send4, ga_hbm, gb_hbm,
             idxr, idxs, a0, a1, b0, b1,
             sga0, sga1, sgb0, sgb1, swa0, swa1, swb0, swb1) = refs
        wid = lax.axis_index("c") * NS + lax.axis_index("s")
        row0 = wid * NCHP
        ebase = wid * EPW
        pltpu.sync_copy(recv4.at[pl.ds(row0, NCHP)], idxr)
        pltpu.sync_copy(send4.at[pl.ds(row0, NCHP)], idxs)
        if with_d:
            pltpu.sync_copy(posx, px)
            pltpu.sync_copy(posy, py)
            pltpu.sync_copy(posz, pz)
        bufa = (a0, a1)
        bufb = (b0, b1)
        sga = (sga0, sga1)
        sgb = (sgb0, sgb1)
        swa = (swa0, swa1)
        swb = (swb0, swb1)

        def issue(k, b):
            pltpu.async_copy(a_hbm.at[idxr.at[k]], bufa[b], sga[b])
            pltpu.async_copy(b_hbm.at[idxs.at[k]], bufb[b], sgb[b])

        def dchunk(k):
            for g in range(C // 16):
                rr = idxr[k, pl.ds(g * 16, 16)]
                ss = idxs[k, pl.ds(g * 16, 16)]
                dx = plsc.load_gather(px, [rr]) - plsc.load_gather(px, [ss])
                dy = plsc.load_gather(py, [rr]) - plsc.load_gather(py, [ss])
                dz = plsc.load_gather(pz, [rr]) - plsc.load_gather(pz, [ss])
                dall[k, pl.ds(g * 16, 16)] = dx * dx + dy * dy + dz * dz

        def step(k, b):
            if with_d:
                dchunk(k)
            pltpu.make_async_copy(a_hbm.at[idxr.at[k]], bufa[b], sga[b]).wait()
            pltpu.make_async_copy(b_hbm.at[idxs.at[k]], bufb[b], sgb[b]).wait()
            off = ebase + k * C
            wa = pltpu.async_copy(bufa[b], ga_hbm.at[pl.ds(off, C)], swa[b])
            wb = pltpu.async_copy(bufb[b], gb_hbm.at[pl.ds(off, C)], swb[b])
            wa.wait()
            wb.wait()

            @pl.when(k + 2 < NCH)
            def _():
                issue(k + 2, b)

        issue(0, 0)
        issue(1, 1)

        @pl.loop(0, (NCH - 1) // 2)
        def _(g):
            step(2 * g, 0)
            step(2 * g + 1, 1)

        step(NCH - 1, 0)
        if with_d:
            pltpu.sync_copy(dall, d_hbm.at[pl.ds(row0, NCHP)])

    return pl.kernel(body, out_type=tuple(out_type), mesh=_mesh(),
                     scratch_types=tuple(scratch),
                     compiler_params=pltpu.CompilerParams(
                         needs_layout_passes=False))


def _sc_segsum_body(oute, recv4, zeros, aggp,
                    idxm, e0, e1, shared, sr0, sr1, ss0, ss1):
    cid = lax.axis_index("c")
    sid = lax.axis_index("s")
    wid = cid * NS + sid
    row0 = wid * NCHP
    ebase = wid * EPW
    pltpu.sync_copy(recv4.at[pl.ds(row0, NCHP)], idxm)

    @pl.when(sid < NS - 1)
    def _():
        pltpu.sync_copy(zeros.at[pl.ds(0, RPT)],
                        shared.at[pl.ds(sid * RPT, RPT)])

    @pl.when(sid == NS - 1)
    def _():
        pltpu.sync_copy(zeros, shared.at[pl.ds((NS - 1) * RPT, RPT_LAST)])

    plsc.subcore_barrier()
    bufs = (e0, e1)
    sr = (sr0, sr1)
    ss = (ss0, ss1)

    def issue(k, b):
        pltpu.async_copy(oute.at[pl.ds(ebase + k * C, C)], bufs[b], sr[b])

    def step(k, b):
        pltpu.make_async_copy(
            oute.at[pl.ds(ebase + k * C, C)], bufs[b], sr[b]).wait()
        sc = pltpu.async_copy(bufs[b], shared.at[idxm.at[k]], ss[b], add=True)
        sc.wait()

        @pl.when(k + 2 < NCH)
        def _():
            issue(k + 2, b)

    issue(0, 0)
    issue(1, 1)

    @pl.loop(0, (NCH - 1) // 2)
    def _(g):
        step(2 * g, 0)
        step(2 * g + 1, 1)

    step(NCH - 1, 0)
    plsc.subcore_barrier()

    @pl.when(sid < NS - 1)
    def _():
        pltpu.sync_copy(shared.at[pl.ds(sid * RPT, RPT)],
                        aggp.at[cid, pl.ds(sid * RPT, RPT)])

    @pl.when(sid == NS - 1)
    def _():
        pltpu.sync_copy(shared.at[pl.ds((NS - 1) * RPT, RPT_LAST)],
                        aggp.at[cid, pl.ds((NS - 1) * RPT, RPT_LAST)])


@functools.lru_cache(maxsize=None)
def _make_sc_segsum():
  return pl.kernel(
    _sc_segsum_body,
    out_type=jax.ShapeDtypeStruct((NC, NN, HID), F32),
    mesh=_mesh(),
    scratch_types=(
        pltpu.VMEM((NCHP, C), jnp.int32),
        pltpu.VMEM((C, HID), F32),
        pltpu.VMEM((C, HID), F32),
        pltpu.VMEM_SHARED((NN, HID), F32),
        pltpu.SemaphoreType.DMA,
        pltpu.SemaphoreType.DMA,
        pltpu.SemaphoreType.DMA,
        pltpu.SemaphoreType.DMA,
    ))


def _sc_gather_d(*args):
    return _make_sc_gather(True)(*args)


def _sc_gather(*args):
    return _make_sc_gather(False)(*args)


def _sc_segsum(*args):
    return _make_sc_segsum()(*args)


def _prep_body(z_, embp_, wr_, ws_, x_o, a_o, b_o):
    io = lax.broadcasted_iota(jnp.int32, (BN, 32), 1)
    oh = (z_[...] == io).astype(F32)
    x = jnp.dot(oh, embp_[...], preferred_element_type=F32, precision=HI)
    x_o[...] = x
    a_o[...] = jnp.dot(x, wr_[...], preferred_element_type=F32, precision=HI)
    b_o[...] = jnp.dot(x, ws_[...], preferred_element_type=F32, precision=HI)


_prep = pl.pallas_call(
    _prep_body,
    grid=(NN // BN,),
    in_specs=[
        pl.BlockSpec((BN, 1), lambda i: (i, 0)),
        pl.BlockSpec((32, HID), lambda i: (0, 0)),
        pl.BlockSpec((HID, HID), lambda i: (0, 0)),
        pl.BlockSpec((HID, HID), lambda i: (0, 0)),
    ],
    out_specs=[pl.BlockSpec((BN, HID), lambda i: (i, 0))] * 3,
    out_shape=[jax.ShapeDtypeStruct((NN, HID), F32)] * 3,
)


def _edge_body(ga, gb, dd, wd_, b1_, w2_, b2_, wi_, bi_, out):
    pre = ga[...] + gb[...] + dd[...] * wd_[...] + b1_[...]
    m1 = _silu(pre)
    m2 = jnp.dot(m1, w2_[...], preferred_element_type=F32, precision=HI) + b2_[...]
    m2 = _silu(m2)
    wgt = _sig(jnp.sum(m2 * wi_[...], axis=1, keepdims=True) + bi_[0, 0])
    out[...] = m2 * wgt


_tc_edge = pl.pallas_call(
    _edge_body,
    grid=(NE // BE,),
    in_specs=[
        pl.BlockSpec((BE, HID), lambda i: (i, 0)),
        pl.BlockSpec((BE, HID), lambda i: (i, 0)),
        pl.BlockSpec((BE, 1), lambda i: (i, 0)),
        pl.BlockSpec((1, HID), lambda i: (0, 0)),
        pl.BlockSpec((1, HID), lambda i: (0, 0)),
        pl.BlockSpec((HID, HID), lambda i: (0, 0)),
        pl.BlockSpec((1, HID), lambda i: (0, 0)),
        pl.BlockSpec((1, HID), lambda i: (0, 0)),
        pl.BlockSpec(memory_space=pltpu.SMEM),
    ],
    out_specs=pl.BlockSpec((BE, HID), lambda i: (i, 0)),
    out_shape=jax.ShapeDtypeStruct((NE, HID), F32),
)


def _node_body(x_, g0_, g1_, w1x_, w1a_, nb1_, w2_, nb2_, wrn_, wsn_,
               xo, ao, bo):
    agg = g0_[...] + g1_[...]
    h = (jnp.dot(x_[...], w1x_[...], preferred_element_type=F32, precision=HI)
         + jnp.dot(agg, w1a_[...], preferred_element_type=F32, precision=HI)
         + nb1_[...])
    h = _silu(h)
    xn = jnp.dot(h, w2_[...], preferred_element_type=F32, precision=HI) + nb2_[...]
    xo[...] = xn
    ao[...] = jnp.dot(xn, wrn_[...], preferred_element_type=F32, precision=HI)
    bo[...] = jnp.dot(xn, wsn_[...], preferred_element_type=F32, precision=HI)


_node0 = pl.pallas_call(
    _node_body,
    grid=(NN // BN,),
    in_specs=[
        pl.BlockSpec((BN, HID), lambda i: (i, 0)),
        pl.BlockSpec((BN, HID), lambda i: (i, 0)),
        pl.BlockSpec((BN, HID), lambda i: (i, 0)),
        pl.BlockSpec((HID, HID), lambda i: (0, 0)),
        pl.BlockSpec((HID, HID), lambda i: (0, 0)),
        pl.BlockSpec((1, HID), lambda i: (0, 0)),
        pl.BlockSpec((HID, HID), lambda i: (0, 0)),
        pl.BlockSpec((1, HID), lambda i: (0, 0)),
        pl.BlockSpec((HID, HID), lambda i: (0, 0)),
        pl.BlockSpec((HID, HID), lambda i: (0, 0)),
    ],
    out_specs=[pl.BlockSpec((BN, HID), lambda i: (i, 0))] * 3,
    out_shape=[jax.ShapeDtypeStruct((NN, HID), F32)] * 3,
)


def _node1_body(x_, g0_, g1_, w1x_, w1a_, nb1_, w2_, nb2_, ow1_, ob1_, ow2_,
                ob2_, yo):
    agg = g0_[...] + g1_[...]
    h = (jnp.dot(x_[...], w1x_[...], preferred_element_type=F32, precision=HI)
         + jnp.dot(agg, w1a_[...], preferred_element_type=F32, precision=HI)
         + nb1_[...])
    h = _silu(h)
    xn = jnp.dot(h, w2_[...], preferred_element_type=F32, precision=HI) + nb2_[...]
    t = _silu(jnp.dot(xn, ow1_[...], preferred_element_type=F32, precision=HI)
              + ob1_[...])
    yo[...] = jnp.dot(t, ow2_[...], preferred_element_type=F32, precision=HI) + ob2_[...]


_node1 = pl.pallas_call(
    _node1_body,
    grid=(NN // BN,),
    in_specs=[
        pl.BlockSpec((BN, HID), lambda i: (i, 0)),
        pl.BlockSpec((BN, HID), lambda i: (i, 0)),
        pl.BlockSpec((BN, HID), lambda i: (i, 0)),
        pl.BlockSpec((HID, HID), lambda i: (0, 0)),
        pl.BlockSpec((HID, HID), lambda i: (0, 0)),
        pl.BlockSpec((1, HID), lambda i: (0, 0)),
        pl.BlockSpec((HID, HID), lambda i: (0, 0)),
        pl.BlockSpec((1, HID), lambda i: (0, 0)),
        pl.BlockSpec((HID, HID), lambda i: (0, 0)),
        pl.BlockSpec((1, HID), lambda i: (0, 0)),
        pl.BlockSpec((HID, HID), lambda i: (0, 0)),
        pl.BlockSpec((1, HID), lambda i: (0, 0)),
    ],
    out_specs=pl.BlockSpec((BN, HID), lambda i: (i, 0)),
    out_shape=jax.ShapeDtypeStruct((NN, HID), F32),
)


def _head_body(y_, bt_, ow1_, ob1_, ow2r_, ob2_, out, acc):
    i = pl.program_id(0)

    @pl.when(i == 0)
    def _():
        acc[...] = jnp.zeros_like(acc)

    io = lax.broadcasted_iota(jnp.int32, (BN, HID), 1)
    oh = (bt_[...] == io).astype(F32)
    acc[...] += lax.dot_general(oh, y_[...], (((0,), (0,)), ((), ())),
                                preferred_element_type=F32, precision=HI)

    @pl.when(i == pl.num_programs(0) - 1)
    def _():
        yg = acc[0:NG, :]
        t = _silu(jnp.dot(yg, ow1_[...], preferred_element_type=F32,
                          precision=HI) + ob1_[...])
        out[...] = (jnp.sum(t * ow2r_[...], axis=1, keepdims=True)
                    + ob2_[0, 0])


_head = pl.pallas_call(
    _head_body,
    grid=(NN // BN,),
    in_specs=[
        pl.BlockSpec((BN, HID), lambda i: (i, 0)),
        pl.BlockSpec((BN, 1), lambda i: (i, 0)),
        pl.BlockSpec((HID, HID), lambda i: (0, 0)),
        pl.BlockSpec((1, HID), lambda i: (0, 0)),
        pl.BlockSpec((1, HID), lambda i: (0, 0)),
        pl.BlockSpec(memory_space=pltpu.SMEM),
    ],
    out_specs=pl.BlockSpec((NG, 1), lambda i: (0, 0)),
    out_shape=jax.ShapeDtypeStruct((NG, 1), F32),
    scratch_shapes=[pltpu.VMEM((HID, HID), F32)],
)


def kernel(z, edge_index, pos, batch, emb,
           e0_w1, e0_b1, e0_w2, e0_b2, e0_wi, e0_bi,
           n0_w1, n0_b1, n0_w2, n0_b2,
           e1_w1, e1_b1, e1_w2, e1_b2, e1_wi, e1_bi,
           n1_w1, n1_b1, n1_w2, n1_b2,
           o1_w1, o1_b1, o1_w2, o1_b2,
           o2_w1, o2_b1, o2_w2, o2_b2):
    pad = lambda v: jnp.pad(v.reshape(NW, NCH, C), ((0, 0), (0, NCHP - NCH), (0, 0))).reshape(NW * NCHP, C)
    recv4 = pad(edge_index[0])
    send4 = pad(edge_index[1])
    posx, posy, posz = pos[:, 0], pos[:, 1], pos[:, 2]
    embp = jnp.zeros((32, HID), F32).at[:20, :].set(emb)
    z2 = z.reshape(NN, 1)
    bt2 = batch.reshape(NN, 1)
    zeros = jnp.zeros((RPT_LAST, HID), F32)
    row = lambda v: v.reshape(1, -1)

    wr0, ws0, wd0 = e0_w1[:HID], e0_w1[HID:2 * HID], e0_w1[2 * HID:]
    wr1, ws1, wd1 = e1_w1[:HID], e1_w1[HID:2 * HID], e1_w1[2 * HID:]

    x, a, b = _prep(z2, embp, wr0, ws0)
    ga, gb, d4 = _sc_gather_d(a, b, recv4, send4, posx, posy, posz)
    d2 = d4.reshape(NW, NCHP, C)[:, :NCH].reshape(NE, 1)

    m = _tc_edge(ga, gb, d2, wd0, row(e0_b1), e0_w2, row(e0_b2),
                 row(e0_wi), e0_bi.reshape(1, 1))
    aggp = _sc_segsum(m, recv4, zeros)
    x, a, b = _node0(x, aggp[0], aggp[1], n0_w1[:HID], n0_w1[HID:],
                     row(n0_b1), n0_w2, row(n0_b2), wr1, ws1)

    ga, gb = _sc_gather(a, b, recv4, send4)
    m = _tc_edge(ga, gb, d2, wd1, row(e1_b1), e1_w2, row(e1_b2),
                 row(e1_wi), e1_bi.reshape(1, 1))
    aggp = _sc_segsum(m, recv4, zeros)
    y = _node1(x, aggp[0], aggp[1], n1_w1[:HID], n1_w1[HID:], row(n1_b1),
               n1_w2, row(n1_b2), o1_w1, row(o1_b1), o1_w2, row(o1_b2))

    return _head(y, bt2, o2_w1, row(o2_b1), row(o2_w2), o2_b2.reshape(1, 1))
